# baseline (speedup 1.0000x reference)
import functools

import jax
import jax.numpy as jnp
from jax import lax
from jax.experimental import pallas as pl
from jax.experimental.pallas import tpu as pltpu
from jax.experimental.pallas import tpu_sc as plsc

N_HERO = 1000000
TEAM = 5
HID = 16
BATCH = 16384

NC = 2
NS = 16
NW = NC * NS
MPW = BATCH // NW
IPW = MPW * TEAM
CHUNK = 128
NCHUNK = IPW // CHUNK
NBLK = MPW // 16

UNIT = 512
NUNIT_FULL = N_HERO // UNIT
REM = N_HERO - NUNIT_FULL * UNIT
N_MAIN = NUNIT_FULL * UNIT
UPW = NUNIT_FULL // NW
NEXTRA = NUNIT_FULL - UPW * NW
NBUF = 4
assert UPW % NBUF == 1


def _tr_body(emb_hbm, out_hbm, stage_v, rows_v, sem_i, sem_o):
    w = lax.axis_index("s") * NC + lax.axis_index("c")
    u0 = w * UPW
    iota = lax.iota(jnp.int32, 16)

    def start_in(u, b):
        return pltpu.async_copy(
            emb_hbm.at[:, pl.ds(u * UNIT, UNIT)], stage_v.at[b], sem_i)

    def start_out(u, b):
        return pltpu.async_copy(
            rows_v.at[b], out_hbm.at[pl.ds(u * (UNIT * HID), UNIT * HID)],
            sem_o)

    iota16 = iota * HID
    bvecs = [jnp.full((16,), b, jnp.int32) for b in range(NBUF)]

    def transpose(b, width):
        UNROLL = 4

        def tstep(c0, carry):
            for u in range(UNROLL):
                c = c0 * UNROLL + u
                base = iota16 + c * (16 * HID)
                for d in range(HID):
                    r = stage_v[b, d, pl.ds(c * 16, 16)]
                    plsc.store_scatter(rows_v, [bvecs[b], base + d], r)
            return carry

        lax.fori_loop(0, width // (16 * UNROLL), tstep, 0)

    def wait_in(b):
        pltpu.make_async_copy(
            emb_hbm.at[:, pl.ds(0, UNIT)], stage_v.at[b], sem_i).wait()

    def wait_out(b):
        pltpu.make_async_copy(
            rows_v.at[b], out_hbm.at[pl.ds(0, UNIT * HID)], sem_o).wait()

    for b in range(NBUF):
        start_in(u0 + b, b)

    def grp(g, carry):
        u = u0 + g * NBUF
        for b in range(NBUF):
            wait_in(b)

            @pl.when(g > 0)
            def _():
                wait_out(b)

            transpose(b, UNIT)
            start_out(u + b, b)
            nxt = u + b + NBUF

            @pl.when(nxt < u0 + UPW)
            def _():
                start_in(nxt, b)
        return carry

    lax.fori_loop(0, UPW // NBUF, grp, 0)

    wait_in(0)
    wait_out(0)
    transpose(0, UNIT)
    start_out(u0 + UPW - 1, 0)
    for b in range(1, NBUF):
        wait_out(b)
    wait_out(0)

    @pl.when(w < NEXTRA)
    def _():
        u = NUNIT_FULL - NEXTRA + w
        pltpu.async_copy(
            emb_hbm.at[:, pl.ds(u * UNIT, UNIT)], stage_v.at[0], sem_i
        ).wait()
        transpose(0, UNIT)
        pltpu.async_copy(
            rows_v.at[0], out_hbm.at[pl.ds(u * (UNIT * HID), UNIT * HID)],
            sem_o).wait()


@functools.partial(jax.jit, donate_argnums=())
def _fm(team, skill, embedding):
    team_flat = team.reshape(-1).astype(jnp.int32)
    skill_flat = skill.reshape(-1)
    emb_t = embedding.T
    tail = embedding[N_MAIN:, :]
    mesh = plsc.VectorSubcoreMesh(
        core_axis_name="c", subcore_axis_name="s", num_cores=NC,
        num_subcores=NS)

    tr = pl.kernel(
        _tr_body,
        out_type=jax.ShapeDtypeStruct((N_MAIN * HID,), jnp.float32),
        mesh=mesh,
        scratch_types=[
            pltpu.VMEM((NBUF, HID, UNIT), jnp.float32),
            pltpu.VMEM((NBUF, UNIT * HID), jnp.float32),
            pltpu.SemaphoreType.DMA,
            pltpu.SemaphoreType.DMA,
        ],
        compiler_params=pltpu.CompilerParams(needs_layout_passes=False),
    )
    emb_lin = tr(emb_t).reshape(N_MAIN, HID)

    f = pl.kernel(
        _fm_gather_body,
        out_type=jax.ShapeDtypeStruct((BATCH,), jnp.float32),
        mesh=mesh,
        scratch_types=[
            pltpu.VMEM((IPW,), jnp.int32),
            pltpu.VMEM((IPW,), jnp.int32),
            pltpu.VMEM((IPW,), jnp.int32),
            pltpu.VMEM((IPW, HID), jnp.float32),
            pltpu.VMEM((IPW,), jnp.float32),
            pltpu.VMEM((REM, HID), jnp.float32),
            pltpu.VMEM((MPW,), jnp.float32),
            pltpu.SemaphoreType.DMA,
            pltpu.SemaphoreType.DMA,
        ],
        compiler_params=pltpu.CompilerParams(
            needs_layout_passes=False, use_tc_tiling_on_sc=False),
    )
    return f(team_flat, skill_flat, emb_lin, tail)


def _fm_gather_body(team_hbm, skill_hbm, emb_hbm, tail_hbm, out_hbm, idx_v,
                    idx2_v, idx3_v, rows_v, sk_v, tail_v, out_v, sem_r,
                    sem_s):
    w = lax.axis_index("s") * NC + lax.axis_index("c")
    base = w * MPW

    pltpu.sync_copy(team_hbm.at[pl.ds(base * TEAM, IPW)], idx_v)
    pltpu.sync_copy(tail_hbm, tail_v)

    iota = lax.iota(jnp.int32, 16)

    def clamp(k, carry):
        sl = pl.ds(k * 16, 16)
        v = idx_v[sl]
        idx2_v[sl] = jnp.minimum(v, N_MAIN - 1)
        idx3_v[sl] = lax.shift_right_logical(v, 1)
        return carry

    lax.fori_loop(0, IPW // 16, clamp, 0)

    copies = []
    for j in range(NCHUNK):
        sl = pl.ds(j * CHUNK, CHUNK)
        copies.append(
            pltpu.async_copy(emb_hbm.at[idx2_v.at[sl]], rows_v.at[sl, :],
                             sem_r))
        copies.append(
            pltpu.async_copy(skill_hbm.at[idx_v.at[sl]], sk_v.at[sl], sem_s))
    for c in copies:
        c.wait()

    def fixup(k, carry):
        sl = pl.ds(k * 16, 16)
        v = idx_v[sl]
        m = v >= N_MAIN

        @pl.when(jnp.any(m))
        def _():
            tidx = jnp.clip(v - N_MAIN, 0, REM - 1)
            rowids = iota + k * 16
            for d in range(HID):
                cold = jnp.full((16,), d, jnp.int32)
                val = plsc.load_gather(tail_v, [tidx, cold], mask=m)
                plsc.store_scatter(rows_v, [rowids, cold], val, mask=m)
        return carry

    lax.fori_loop(0, IPW // 16, fixup, 0)

    iota5 = iota * TEAM
    zf = jnp.zeros((16,), jnp.float32)
    z16 = jnp.zeros((16,), jnp.int32)

    def blk(b, carry):
        rb = b * (16 * TEAM)
        rows = [iota5 + (rb + i) for i in range(TEAM)]
        tsk = zf
        for i in range(TEAM):
            tsk = tsk + plsc.load_gather(sk_v, [rows[i]])
        acc = zf
        for d in range(HID):
            cold = jnp.full((16,), d, jnp.int32)
            e = [plsc.load_gather(rows_v, [rows[i], cold]) for i in range(TEAM)]
            s = e[0] + e[1] + e[2] + e[3] + e[4]
            sq = e[0] * e[0] + e[1] * e[1] + e[2] * e[2] + e[3] * e[3] + e[4] * e[4]
            acc = acc + (s * s - sq)
        out_v[pl.ds(b * 16, 16)] = tsk + 0.5 * acc
        return carry

    lax.fori_loop(0, NBLK, blk, 0)
    pltpu.sync_copy(out_v, out_hbm.at[pl.ds(base, MPW)])


def kernel(team, skill, embedding):
    return _fm(team, skill, embedding).reshape(-1, 1)


# speedup vs baseline: 1.0430x; 1.0430x over previous
import functools

import jax
import jax.numpy as jnp
from jax import lax
from jax.experimental import pallas as pl
from jax.experimental.pallas import tpu as pltpu
from jax.experimental.pallas import tpu_sc as plsc

N_HERO = 1000000
TEAM = 5
HID = 16
BATCH = 16384

NC = 2
NS = 16
NW = NC * NS
MPW = BATCH // NW
IPW = MPW * TEAM
CHUNK = 128
NCHUNK = IPW // CHUNK
NBLK = MPW // 16

UNIT = 512
NUNIT_FULL = N_HERO // UNIT
REM = N_HERO - NUNIT_FULL * UNIT
N_MAIN = NUNIT_FULL * UNIT
UPW = NUNIT_FULL // NW
NEXTRA = NUNIT_FULL - UPW * NW
NBUF = 2
assert UPW % NBUF == 1


def _tr_body(emb_hbm, out_hbm, stage_v, rows_v, sem_i, sem_o):
    w = lax.axis_index("s") * NC + lax.axis_index("c")
    u0 = w * UPW
    iota = lax.iota(jnp.int32, 16)

    def start_in(u, b):
        return pltpu.async_copy(
            emb_hbm.at[:, pl.ds(u * UNIT, UNIT)], stage_v.at[b], sem_i)

    def start_out(u, b):
        return pltpu.async_copy(
            rows_v.at[b], out_hbm.at[pl.ds(u * (UNIT * HID), UNIT * HID)],
            sem_o)

    iota16 = iota * HID
    bvecs = [jnp.full((16,), b, jnp.int32) for b in range(NBUF)]

    def transpose(b, width):
        UNROLL = 1

        def tstep(c0, carry):
            for u in range(UNROLL):
                c = c0 * UNROLL + u
                base = iota16 + c * (16 * HID)
                for d in range(HID):
                    r = stage_v[b, d, pl.ds(c * 16, 16)]
                    plsc.store_scatter(rows_v, [bvecs[b], base + d], r)
            return carry

        lax.fori_loop(0, width // (16 * UNROLL), tstep, 0)

    def wait_in(b):
        pltpu.make_async_copy(
            emb_hbm.at[:, pl.ds(0, UNIT)], stage_v.at[b], sem_i).wait()

    def wait_out(b):
        pltpu.make_async_copy(
            rows_v.at[b], out_hbm.at[pl.ds(0, UNIT * HID)], sem_o).wait()

    for b in range(NBUF):
        start_in(u0 + b, b)

    def grp(g, carry):
        u = u0 + g * NBUF
        for b in range(NBUF):
            wait_in(b)

            @pl.when(g > 0)
            def _():
                wait_out(b)

            transpose(b, UNIT)
            start_out(u + b, b)
            nxt = u + b + NBUF

            @pl.when(nxt < u0 + UPW)
            def _():
                start_in(nxt, b)
        return carry

    lax.fori_loop(0, UPW // NBUF, grp, 0)

    wait_in(0)
    wait_out(0)
    transpose(0, UNIT)
    start_out(u0 + UPW - 1, 0)
    for b in range(1, NBUF):
        wait_out(b)
    wait_out(0)

    @pl.when(w < NEXTRA)
    def _():
        u = NUNIT_FULL - NEXTRA + w
        pltpu.async_copy(
            emb_hbm.at[:, pl.ds(u * UNIT, UNIT)], stage_v.at[0], sem_i
        ).wait()
        transpose(0, UNIT)
        pltpu.async_copy(
            rows_v.at[0], out_hbm.at[pl.ds(u * (UNIT * HID), UNIT * HID)],
            sem_o).wait()


@functools.partial(jax.jit, donate_argnums=())
def _fm(team, skill, embedding):
    team_flat = team.reshape(-1).astype(jnp.int32)
    skill_flat = skill.reshape(-1)
    emb_t = embedding.T
    tail = embedding[N_MAIN:, :]
    mesh = plsc.VectorSubcoreMesh(
        core_axis_name="c", subcore_axis_name="s", num_cores=NC,
        num_subcores=NS)

    tr = pl.kernel(
        _tr_body,
        out_type=jax.ShapeDtypeStruct((N_MAIN * HID,), jnp.float32),
        mesh=mesh,
        scratch_types=[
            pltpu.VMEM((NBUF, HID, UNIT), jnp.float32),
            pltpu.VMEM((NBUF, UNIT * HID), jnp.float32),
            pltpu.SemaphoreType.DMA,
            pltpu.SemaphoreType.DMA,
        ],
        compiler_params=pltpu.CompilerParams(needs_layout_passes=False),
    )
    emb_lin = tr(emb_t).reshape(N_MAIN, HID)

    f = pl.kernel(
        _fm_gather_body,
        out_type=jax.ShapeDtypeStruct((BATCH,), jnp.float32),
        mesh=mesh,
        scratch_types=[
            pltpu.VMEM((IPW,), jnp.int32),
            pltpu.VMEM((IPW,), jnp.int32),
            pltpu.VMEM((IPW, HID), jnp.float32),
            pltpu.VMEM((IPW,), jnp.float32),
            pltpu.VMEM((REM, HID), jnp.float32),
            pltpu.VMEM((MPW,), jnp.float32),
            pltpu.SemaphoreType.DMA,
            pltpu.SemaphoreType.DMA,
        ],
        compiler_params=pltpu.CompilerParams(
            needs_layout_passes=False, use_tc_tiling_on_sc=False),
    )
    return f(team_flat, skill_flat, emb_lin, tail)


def _fm_gather_body(team_hbm, skill_hbm, emb_hbm, tail_hbm, out_hbm, idx_v,
                    idx2_v, rows_v, sk_v, tail_v, out_v, sem_r, sem_s):
    w = lax.axis_index("s") * NC + lax.axis_index("c")
    base = w * MPW

    pltpu.sync_copy(team_hbm.at[pl.ds(base * TEAM, IPW)], idx_v)
    pltpu.sync_copy(tail_hbm, tail_v)

    iota = lax.iota(jnp.int32, 16)

    def clamp(k, carry):
        sl = pl.ds(k * 16, 16)
        idx2_v[sl] = jnp.minimum(idx_v[sl], N_MAIN - 1)
        return carry

    lax.fori_loop(0, IPW // 16, clamp, 0)

    copies = []
    for j in range(NCHUNK):
        sl = pl.ds(j * CHUNK, CHUNK)
        copies.append(
            pltpu.async_copy(emb_hbm.at[idx2_v.at[sl]], rows_v.at[sl, :],
                             sem_r))
        copies.append(
            pltpu.async_copy(skill_hbm.at[idx_v.at[sl]], sk_v.at[sl], sem_s))
    for c in copies:
        c.wait()

    def fixup(k, carry):
        sl = pl.ds(k * 16, 16)
        v = idx_v[sl]
        m = v >= N_MAIN

        @pl.when(jnp.any(m))
        def _():
            tidx = jnp.clip(v - N_MAIN, 0, REM - 1)
            rowids = iota + k * 16
            for d in range(HID):
                cold = jnp.full((16,), d, jnp.int32)
                val = plsc.load_gather(tail_v, [tidx, cold], mask=m)
                plsc.store_scatter(rows_v, [rowids, cold], val, mask=m)
        return carry

    lax.fori_loop(0, IPW // 16, fixup, 0)

    iota5 = iota * TEAM
    zf = jnp.zeros((16,), jnp.float32)
    z16 = jnp.zeros((16,), jnp.int32)

    def blk(b, carry):
        rb = b * (16 * TEAM)
        rows = [iota5 + (rb + i) for i in range(TEAM)]
        tsk = zf
        for i in range(TEAM):
            tsk = tsk + plsc.load_gather(sk_v, [rows[i]])
        acc = zf
        for d in range(HID):
            cold = jnp.full((16,), d, jnp.int32)
            e = [plsc.load_gather(rows_v, [rows[i], cold]) for i in range(TEAM)]
            s = e[0] + e[1] + e[2] + e[3] + e[4]
            sq = e[0] * e[0] + e[1] * e[1] + e[2] * e[2] + e[3] * e[3] + e[4] * e[4]
            acc = acc + (s * s - sq)
        out_v[pl.ds(b * 16, 16)] = tsk + 0.5 * acc
        return carry

    lax.fori_loop(0, NBLK, blk, 0)
    pltpu.sync_copy(out_v, out_hbm.at[pl.ds(base, MPW)])


def kernel(team, skill, embedding):
    return _fm(team, skill, embedding).reshape(-1, 1)


# speedup vs baseline: 1.0434x; 1.0004x over previous
import functools

import jax
import jax.numpy as jnp
from jax import lax
from jax.experimental import pallas as pl
from jax.experimental.pallas import tpu as pltpu
from jax.experimental.pallas import tpu_sc as plsc

N_HERO = 1000000
TEAM = 5
HID = 16
BATCH = 16384

NC = 2
NS = 16
NW = NC * NS
MPW = BATCH // NW
IPW = MPW * TEAM
CHUNK = 128
NCHUNK = IPW // CHUNK
NBLK = MPW // 16

UNIT = 512
NUNIT_FULL = N_HERO // UNIT
REM = N_HERO - NUNIT_FULL * UNIT
N_MAIN = NUNIT_FULL * UNIT
UPW = NUNIT_FULL // NW
NEXTRA = NUNIT_FULL - UPW * NW
NBUF = 2
assert UPW % NBUF == 1


def _tr_body(emb_hbm, out_hbm, stage_v, rows_v, sem_i, sem_o):
    w = lax.axis_index("s") * NC + lax.axis_index("c")
    u0 = w * UPW
    iota = lax.iota(jnp.int32, 16)

    def start_in(u, b):
        return pltpu.async_copy(
            emb_hbm.at[:, pl.ds(u * UNIT, UNIT)], stage_v.at[b], sem_i)

    def start_out(u, b):
        return pltpu.async_copy(
            rows_v.at[b], out_hbm.at[pl.ds(u * (UNIT * HID), UNIT * HID)],
            sem_o)

    iota16 = iota * HID
    bvecs = [jnp.full((16,), b, jnp.int32) for b in range(NBUF)]

    def transpose(b, width):
        UNROLL = 1

        def tstep(c0, carry):
            for u in range(UNROLL):
                c = c0 * UNROLL + u
                base = iota16 + c * (16 * HID)
                for d in range(HID):
                    r = stage_v[b, d, pl.ds(c * 16, 16)]
                    plsc.store_scatter(rows_v, [bvecs[b], base + d], r)
            return carry

        lax.fori_loop(0, width // (16 * UNROLL), tstep, 0)

    def wait_in(b):
        pltpu.make_async_copy(
            emb_hbm.at[:, pl.ds(0, UNIT)], stage_v.at[b], sem_i).wait()

    def wait_out(b):
        pltpu.make_async_copy(
            rows_v.at[b], out_hbm.at[pl.ds(0, UNIT * HID)], sem_o).wait()

    for b in range(NBUF):
        start_in(u0 + b, b)

    def grp(g, carry):
        u = u0 + g * NBUF
        for b in range(NBUF):
            wait_in(b)

            @pl.when(g > 0)
            def _():
                wait_out(b)

            transpose(b, UNIT)
            start_out(u + b, b)
            nxt = u + b + NBUF

            @pl.when(nxt < u0 + UPW)
            def _():
                start_in(nxt, b)
        return carry

    lax.fori_loop(0, UPW // NBUF, grp, 0)

    wait_in(0)
    wait_out(0)
    transpose(0, UNIT)
    start_out(u0 + UPW - 1, 0)
    for b in range(1, NBUF):
        wait_out(b)
    wait_out(0)

    @pl.when(w < NEXTRA)
    def _():
        u = NUNIT_FULL - NEXTRA + w
        pltpu.async_copy(
            emb_hbm.at[:, pl.ds(u * UNIT, UNIT)], stage_v.at[0], sem_i
        ).wait()
        transpose(0, UNIT)
        pltpu.async_copy(
            rows_v.at[0], out_hbm.at[pl.ds(u * (UNIT * HID), UNIT * HID)],
            sem_o).wait()


@functools.partial(jax.jit, donate_argnums=())
def _fm(team, skill, embedding):
    team_flat = team.reshape(-1).astype(jnp.int32)
    skill_flat = skill.reshape(-1)
    emb_t = embedding.T
    tail = embedding[N_MAIN:, :]
    mesh = plsc.VectorSubcoreMesh(
        core_axis_name="c", subcore_axis_name="s", num_cores=NC,
        num_subcores=NS)

    tr = pl.kernel(
        _tr_body,
        out_type=jax.ShapeDtypeStruct((N_MAIN * HID,), jnp.float32),
        mesh=mesh,
        scratch_types=[
            pltpu.VMEM((NBUF, HID, UNIT), jnp.float32),
            pltpu.VMEM((NBUF, UNIT * HID), jnp.float32),
            pltpu.SemaphoreType.DMA,
            pltpu.SemaphoreType.DMA,
        ],
        compiler_params=pltpu.CompilerParams(needs_layout_passes=False),
    )
    emb_lin = tr(emb_t).reshape(N_MAIN, HID)

    f = pl.kernel(
        _fm_gather_body,
        out_type=jax.ShapeDtypeStruct((BATCH,), jnp.float32),
        mesh=mesh,
        scratch_types=[
            pltpu.VMEM((IPW,), jnp.int32),
            pltpu.VMEM((IPW,), jnp.int32),
            pltpu.VMEM((IPW, HID), jnp.float32),
            pltpu.VMEM((IPW,), jnp.float32),
            pltpu.VMEM((REM, HID), jnp.float32),
            pltpu.VMEM((MPW,), jnp.float32),
            pltpu.SemaphoreType.DMA,
            pltpu.SemaphoreType.DMA,
        ],
        compiler_params=pltpu.CompilerParams(
            needs_layout_passes=False, use_tc_tiling_on_sc=False),
    )
    return f(team_flat, skill_flat, emb_lin, tail)


def _fm_gather_body(team_hbm, skill_hbm, emb_hbm, tail_hbm, out_hbm, idx_v,
                    idx2_v, rows_v, sk_v, tail_v, out_v, sem_r, sem_s):
    w = lax.axis_index("s") * NC + lax.axis_index("c")
    base = w * MPW

    pltpu.sync_copy(team_hbm.at[pl.ds(base * TEAM, IPW)], idx_v)
    pltpu.sync_copy(tail_hbm, tail_v)

    iota = lax.iota(jnp.int32, 16)

    def clamp(k, carry):
        sl = pl.ds(k * 16, 16)
        idx2_v[sl] = jnp.minimum(idx_v[sl], N_MAIN - 1)
        return carry

    lax.fori_loop(0, IPW // 16, clamp, 0)

    copies = []
    for j in range(NCHUNK):
        sl = pl.ds(j * CHUNK, CHUNK)
        copies.append(
            pltpu.async_copy(emb_hbm.at[idx2_v.at[sl]], rows_v.at[sl, :],
                             sem_r))
        copies.append(
            pltpu.async_copy(skill_hbm.at[idx_v.at[sl]], sk_v.at[sl], sem_s))
    for c in copies:
        c.wait()

    def fixup(k, carry):
        sl = pl.ds(k * 16, 16)
        v = idx_v[sl]
        m = v >= N_MAIN

        @pl.when(jnp.any(m))
        def _():
            tidx = jnp.clip(v - N_MAIN, 0, REM - 1)
            rowids = iota + k * 16
            for d in range(HID):
                cold = jnp.full((16,), d, jnp.int32)
                val = plsc.load_gather(tail_v, [tidx, cold], mask=m)
                plsc.store_scatter(rows_v, [rowids, cold], val, mask=m)
        return carry

    lax.fori_loop(0, IPW // 16, fixup, 0)

    iota5 = iota * TEAM
    zf = jnp.zeros((16,), jnp.float32)

    def blk(b, carry):
        rb = b * (16 * TEAM)
        rows = [iota5 + (rb + i) for i in range(TEAM)]
        tsk = zf
        for i in range(TEAM):
            tsk = tsk + plsc.load_gather(sk_v, [rows[i]])
        acc = zf
        for d in range(HID):
            cold = jnp.full((16,), d, jnp.int32)
            e = [plsc.load_gather(rows_v, [rows[i], cold]) for i in range(TEAM)]
            s = e[0] + e[1] + e[2] + e[3] + e[4]
            sq = e[0] * e[0] + e[1] * e[1] + e[2] * e[2] + e[3] * e[3] + e[4] * e[4]
            acc = acc + (s * s - sq)
        out_v[pl.ds(b * 16, 16)] = tsk + 0.5 * acc
        return carry

    lax.fori_loop(0, NBLK, blk, 0)
    pltpu.sync_copy(out_v, out_hbm.at[pl.ds(base, MPW)])


def kernel(team, skill, embedding):
    return _fm(team, skill, embedding).reshape(-1, 1)
